# baseline (speedup 1.0000x reference)
import functools

import jax
import jax.numpy as jnp
from jax import lax
from jax.experimental import pallas as pl
from jax.experimental.pallas import tpu as pltpu
from jax.experimental.pallas import tpu_sc as plsc

N = 10000
E = 320000
D = 128
H = 64
HH = H // 2
R = 3

NC = 2
NS = 16
NW = NC * NS

CH = 128
EPT1 = 20480
EPW2 = 10240
NCH1 = EPT1 // CH
NCH2 = EPW2 // CH

_MESH = dict(core_axis_name="c", subcore_axis_name="s", num_cores=NC,
             num_subcores=NS)
_SC_PARAMS = pltpu.CompilerParams(use_tc_tiling_on_sc=False,
                                  needs_layout_passes=False)


def _tc_rel_transform(x, W_rel):
    def body(x_ref, w_ref, o_ref):
        xb = x_ref[...]
        for r in range(R):
            mm = jnp.dot(xb, w_ref[r], preferred_element_type=jnp.float32)
            o_ref[0, r] = mm[:, :HH]
            o_ref[1, r] = mm[:, HH:]

    return pl.pallas_call(
        body,
        grid=(10,),
        in_specs=[pl.BlockSpec((N // 10, D), lambda i: (i, 0)),
                  pl.BlockSpec((R, D, H), lambda i: (0, 0, 0))],
        out_specs=pl.BlockSpec((NC, R, N // 10, HH), lambda i: (0, 0, i, 0)),
        out_shape=jax.ShapeDtypeStruct((NC, R, N, HH), jnp.float32),
    )(x, W_rel)


def _tc_mid(parts, hist, x, W_root, b1, Wg_nbr):
    def body(p_ref, h_ref, x_ref, wr_ref, b1_ref, wn_ref, oh_ref, ohw_ref):
        cnt = jnp.sum(h_ref[...], axis=0)
        norm = 2.0 / jnp.maximum(cnt, 2.0)
        p = jnp.concatenate([p_ref[0], p_ref[1]], axis=2)
        agg = (p[0] * norm[:, 0:1] + p[1] * norm[:, 1:2] + p[2] * norm[:, 2:3])
        h = agg + jnp.dot(x_ref[...], wr_ref[...],
                          preferred_element_type=jnp.float32) + b1_ref[...]
        h = jnp.maximum(h, 0.0)
        oh_ref[...] = h
        ohw_ref[...] = jnp.dot(h, wn_ref[...],
                               preferred_element_type=jnp.float32)

    B = N // 10
    return pl.pallas_call(
        body,
        grid=(10,),
        in_specs=[pl.BlockSpec((NC, R, B, HH), lambda i: (0, 0, i, 0)),
                  pl.BlockSpec((NW, B, R), lambda i: (0, i, 0)),
                  pl.BlockSpec((B, D), lambda i: (i, 0)),
                  pl.BlockSpec((D, H), lambda i: (0, 0)),
                  pl.BlockSpec((1, H), lambda i: (0, 0)),
                  pl.BlockSpec((H, H), lambda i: (0, 0))],
        out_specs=[pl.BlockSpec((B, H), lambda i: (i, 0)),
                   pl.BlockSpec((B, H), lambda i: (i, 0))],
        out_shape=[jax.ShapeDtypeStruct((N, H), jnp.float32),
                   jax.ShapeDtypeStruct((N, H), jnp.float32)],
    )(parts, hist, x, W_root, b1, Wg_nbr)


def _tc_post(x, h, parts2, Wg_root, b2):
    def body(x_ref, h_ref, q_ref, wg_ref, b2_ref, o_ref):
        nbrw = q_ref[0] + q_ref[1]
        h2 = jnp.dot(h_ref[...], wg_ref[...],
                     preferred_element_type=jnp.float32) + nbrw + b2_ref[...]
        h2 = jnp.maximum(h2, 0.0)
        o_ref[...] = jnp.concatenate([x_ref[...], h2], axis=1)

    B = N // 10
    return pl.pallas_call(
        body,
        grid=(10,),
        in_specs=[pl.BlockSpec((B, D), lambda i: (i, 0)),
                  pl.BlockSpec((B, H), lambda i: (i, 0)),
                  pl.BlockSpec((NC, B, H), lambda i: (0, i, 0)),
                  pl.BlockSpec((H, H), lambda i: (0, 0)),
                  pl.BlockSpec((1, H), lambda i: (0, 0))],
        out_specs=pl.BlockSpec((B, D + H), lambda i: (i, 0)),
        out_shape=jax.ShapeDtypeStruct((N, D + H), jnp.float32),
    )(x, h, parts2, Wg_root, b2)


def _sc_pass1(table, gsrc2, gdst, hidx, zrows, zhist):
    n_rows = R * N
    rpt = n_rows // NS

    @functools.partial(
        pl.kernel,
        out_type=(jax.ShapeDtypeStruct((NC, n_rows, HH), jnp.float32),
                  jax.ShapeDtypeStruct((NW * n_rows,), jnp.float32)),
        mesh=plsc.VectorSubcoreMesh(**_MESH),
        compiler_params=_SC_PARAMS,
        scratch_types=[
            pltpu.VMEM_SHARED((n_rows + NS, HH), jnp.float32),
            pltpu.VMEM((8, CH), jnp.int32),
            pltpu.VMEM((8, CH), jnp.int32),
            pltpu.VMEM((8, CH), jnp.int32),
            pltpu.VMEM((4, CH, HH), jnp.float32),
            pltpu.VMEM((n_rows + NS,), jnp.float32),
            pltpu.SemaphoreType.DMA((8,)),
            pltpu.SemaphoreType.DMA((4,)),
            pltpu.SemaphoreType.DMA((4,)),
        ],
    )
    def k(tab_hbm, gsrc_hbm, gdst_hbm, hidx_hbm, z_hbm, zh_hbm,
          parts_hbm, hist_hbm,
          acc, si_v, di_v, hi_v, rows_v, hist_v, sem_i, sem_g, sem_s):
        c = lax.axis_index("c")
        s = lax.axis_index("s")
        wid = c * NS + s
        gbase = (c * NS + s) * EPT1
        dbase = s * EPT1

        def idx_start(j, q):
            pltpu.async_copy(gsrc_hbm.at[pl.ds(gbase + j * CH, CH)],
                             si_v.at[q], sem_i.at[q])
            pltpu.async_copy(gdst_hbm.at[pl.ds(dbase + j * CH, CH)],
                             di_v.at[q], sem_i.at[q])
            pltpu.async_copy(hidx_hbm.at[pl.ds(dbase + j * CH, CH)],
                             hi_v.at[q], sem_i.at[q])

        def idx_wait(j, q):
            pltpu.make_async_copy(gsrc_hbm.at[pl.ds(gbase + j * CH, CH)],
                                  si_v.at[q], sem_i.at[q]).wait()
            pltpu.make_async_copy(gdst_hbm.at[pl.ds(dbase + j * CH, CH)],
                                  di_v.at[q], sem_i.at[q]).wait()
            pltpu.make_async_copy(hidx_hbm.at[pl.ds(dbase + j * CH, CH)],
                                  hi_v.at[q], sem_i.at[q]).wait()

        def scatter_start(rq, iq):
            pltpu.async_copy(rows_v.at[rq], acc.at[di_v.at[iq]],
                             sem_s.at[rq], add=True)

        def scatter_wait(rq, iq):
            pltpu.make_async_copy(rows_v.at[rq], acc.at[di_v.at[iq]],
                                  sem_s.at[rq]).wait()

        def gather_start(rq, iq):
            pltpu.async_copy(tab_hbm.at[si_v.at[iq]], rows_v.at[rq],
                             sem_g.at[rq])

        def gather_wait(rq, iq):
            pltpu.make_async_copy(tab_hbm.at[si_v.at[iq]], rows_v.at[rq],
                                  sem_g.at[rq]).wait()

        ones = jnp.ones((16,), jnp.float32)

        def hist_add(q):
            for g in range(CH // 16):
                plsc.addupdate_scatter(hist_v,
                                       [hi_v[q, pl.ds(g * 16, 16)]], ones)

        pltpu.sync_copy(z_hbm, acc.at[pl.ds(s * rpt, rpt)])
        pltpu.sync_copy(zh_hbm, hist_v)
        idx_start(0, 0)
        idx_start(1, 1)
        plsc.subcore_barrier()

        def octet(t, carry):
            for b in range(8):
                j = 8 * t + b
                rb = b % 4

                @pl.when(j >= 4)
                def _():
                    scatter_wait(rb, (b + 4) % 8)

                @pl.when(j + 2 < NCH1)
                def _():
                    idx_start(j + 2, (b + 2) % 8)

                @pl.when(j >= 1)
                def _():
                    gather_wait((rb + 3) % 4, (b + 7) % 8)
                    scatter_start((rb + 3) % 4, (b + 7) % 8)

                idx_wait(j, b)
                gather_start(rb, b)
                if b == 0:
                    @pl.when(j >= 1)
                    def _():
                        hist_add(7)
                else:
                    hist_add(b - 1)
            return carry

        lax.fori_loop(0, NCH1 // 8, octet, 0)
        gather_wait(3, 7)
        scatter_start(3, 7)
        hist_add(7)
        for kk in range(4):
            scatter_wait(kk, 4 + kk)
        plsc.subcore_barrier()
        pltpu.sync_copy(acc.at[pl.ds(s * rpt, rpt)],
                        parts_hbm.at[c, pl.ds(s * rpt, rpt)])
        pltpu.sync_copy(hist_v.at[pl.ds(0, n_rows)],
                        hist_hbm.at[pl.ds(wid * n_rows, n_rows)])

    return k(table, gsrc2, gdst, hidx, zrows, zhist)


def _sc_pass2(table, gsrc, gdst, zrows):
    n_rows = N
    rpt = n_rows // NS

    @functools.partial(
        pl.kernel,
        out_type=jax.ShapeDtypeStruct((NC, n_rows, H), jnp.float32),
        mesh=plsc.VectorSubcoreMesh(**_MESH),
        compiler_params=_SC_PARAMS,
        scratch_types=[
            pltpu.VMEM_SHARED((n_rows + NS, H), jnp.float32),
            pltpu.VMEM((8, CH), jnp.int32),
            pltpu.VMEM((8, CH), jnp.int32),
            pltpu.VMEM((4, CH, H), jnp.float32),
            pltpu.SemaphoreType.DMA((8,)),
            pltpu.SemaphoreType.DMA((4,)),
            pltpu.SemaphoreType.DMA((4,)),
        ],
    )
    def k(tab_hbm, gsrc_hbm, gdst_hbm, z_hbm, parts_hbm,
          acc, si_v, di_v, rows_v, sem_i, sem_g, sem_s):
        c = lax.axis_index("c")
        s = lax.axis_index("s")
        base = (c * NS + s) * EPW2

        def idx_start(j, q):
            pltpu.async_copy(gsrc_hbm.at[pl.ds(base + j * CH, CH)],
                             si_v.at[q], sem_i.at[q])
            pltpu.async_copy(gdst_hbm.at[pl.ds(base + j * CH, CH)],
                             di_v.at[q], sem_i.at[q])

        def idx_wait(j, q):
            pltpu.make_async_copy(gsrc_hbm.at[pl.ds(base + j * CH, CH)],
                                  si_v.at[q], sem_i.at[q]).wait()
            pltpu.make_async_copy(gdst_hbm.at[pl.ds(base + j * CH, CH)],
                                  di_v.at[q], sem_i.at[q]).wait()

        def scatter_start(rq, iq):
            pltpu.async_copy(rows_v.at[rq], acc.at[di_v.at[iq]],
                             sem_s.at[rq], add=True)

        def scatter_wait(rq, iq):
            pltpu.make_async_copy(rows_v.at[rq], acc.at[di_v.at[iq]],
                                  sem_s.at[rq]).wait()

        def gather_start(rq, iq):
            pltpu.async_copy(tab_hbm.at[si_v.at[iq]], rows_v.at[rq],
                             sem_g.at[rq])

        def gather_wait(rq, iq):
            pltpu.make_async_copy(tab_hbm.at[si_v.at[iq]], rows_v.at[rq],
                                  sem_g.at[rq]).wait()

        pltpu.sync_copy(z_hbm, acc.at[pl.ds(s * rpt, rpt)])
        idx_start(0, 0)
        idx_start(1, 1)
        plsc.subcore_barrier()

        def octet(t, carry):
            for b in range(8):
                j = 8 * t + b
                rb = b % 4

                @pl.when(j >= 4)
                def _():
                    scatter_wait(rb, (b + 4) % 8)

                @pl.when(j + 2 < NCH2)
                def _():
                    idx_start(j + 2, (b + 2) % 8)

                @pl.when(j >= 1)
                def _():
                    gather_wait((rb + 3) % 4, (b + 7) % 8)
                    scatter_start((rb + 3) % 4, (b + 7) % 8)

                idx_wait(j, b)
                gather_start(rb, b)
            return carry

        lax.fori_loop(0, NCH2 // 8, octet, 0)
        gather_wait(3, 7)
        scatter_start(3, 7)
        for kk in range(4):
            scatter_wait(kk, 4 + kk)
        plsc.subcore_barrier()
        pltpu.sync_copy(acc.at[pl.ds(s * rpt, rpt)],
                        parts_hbm.at[c, pl.ds(s * rpt, rpt)])

    return k(table, gsrc, gdst, zrows)


def kernel(x, edge_index, edge_type, W_rel, W_root, b1, Wg_root, Wg_nbr, b2):
    src = edge_index[0]
    dst = edge_index[1]

    def pad16(idx, trash):
        ppw = EPT1 - E // NS
        if trash is None:
            fill = jnp.zeros((NS, ppw), jnp.int32)
        else:
            fill = jnp.broadcast_to(trash + jnp.arange(NS)[:, None],
                                    (NS, ppw)).astype(jnp.int32)
        return jnp.concatenate([idx.reshape(NS, E // NS), fill],
                               axis=1).ravel()

    def pad32(idx, trash):
        ppw = EPW2 - E // NW
        if trash is None:
            fill = jnp.zeros((NW, ppw), jnp.int32)
        else:
            fill = jnp.broadcast_to(
                trash + (jnp.arange(NW)[:, None] % NS),
                (NW, ppw)).astype(jnp.int32)
        return jnp.concatenate([idx.reshape(NW, E // NW), fill],
                               axis=1).ravel()

    gsrc1 = pad16(edge_type * N + src, None)
    gsrc2 = jnp.concatenate([gsrc1, gsrc1 + R * N])
    gdst1 = pad16(edge_type * N + dst, R * N)
    hidx1 = pad16(dst * R + edge_type, R * N)
    src_p = pad32(src, None)
    dst_p = pad32(dst, N)

    zrows1 = jnp.zeros((R * N // NS, HH), jnp.float32)
    zrows2 = jnp.zeros((N // NS, H), jnp.float32)
    zhist = jnp.zeros((R * N + NS,), jnp.float32)

    xr = _tc_rel_transform(x, W_rel)
    parts, hist = _sc_pass1(xr.reshape(NC * R * N, HH), gsrc2, gdst1,
                            hidx1, zrows1, zhist)
    h, hw = _tc_mid(parts.reshape(NC, R, N, HH), hist.reshape(NW, N, R),
                    x, W_root, b1.reshape(1, H), Wg_nbr)
    parts2 = _sc_pass2(hw, src_p, dst_p, zrows2)
    out = _tc_post(x, h, parts2, Wg_root, b2.reshape(1, H))
    return out


# speedup vs baseline: 1.0847x; 1.0847x over previous
import functools

import jax
import jax.numpy as jnp
from jax import lax
from jax.experimental import pallas as pl
from jax.experimental.pallas import tpu as pltpu
from jax.experimental.pallas import tpu_sc as plsc

N = 10000
E = 320000
D = 128
H = 64
HH = H // 2
R = 3

NC = 2
NS = 16
NW = NC * NS

CH = 128
EPT1 = 20480
EPW2 = 10240
NCH1 = EPT1 // CH
NCH2 = EPW2 // CH

_MESH = dict(core_axis_name="c", subcore_axis_name="s", num_cores=NC,
             num_subcores=NS)
_SC_PARAMS = pltpu.CompilerParams(use_tc_tiling_on_sc=False,
                                  needs_layout_passes=False)


NP2 = N // 2
BP = NP2 // 5


def _tc_rel_transform(xp, W_rel):
    def body(x_ref, w_ref, o_ref):
        xb = x_ref[...]
        for r in range(R):
            me = jnp.dot(xb[:, :D], w_ref[r], preferred_element_type=jnp.float32)
            mo = jnp.dot(xb[:, D:], w_ref[r], preferred_element_type=jnp.float32)
            o_ref[0, r] = jnp.concatenate([me[:, :HH], mo[:, :HH]], axis=1)
            o_ref[1, r] = jnp.concatenate([me[:, HH:], mo[:, HH:]], axis=1)

    return pl.pallas_call(
        body,
        grid=(5,),
        in_specs=[pl.BlockSpec((BP, 2 * D), lambda i: (i, 0)),
                  pl.BlockSpec((R, D, H), lambda i: (0, 0, 0))],
        out_specs=pl.BlockSpec((NC, R, BP, H), lambda i: (0, 0, i, 0)),
        out_shape=jax.ShapeDtypeStruct((NC, R, NP2, H), jnp.float32),
    )(xp, W_rel)


def _tc_mid(parts, hist, xp, W_root, b1, Wg_nbr):
    def body(p_ref, h_ref, x_ref, wr_ref, b1_ref, wn_ref, oh_ref, ohw_ref):
        cnt = jnp.sum(h_ref[...], axis=0)
        norm = 2.0 / jnp.maximum(cnt, 2.0)
        aggp = []
        for c in range(NC):
            acc = None
            for r in range(R):
                scale = jnp.concatenate(
                    [jnp.broadcast_to(norm[:, r:r + 1], (BP, HH)),
                     jnp.broadcast_to(norm[:, R + r:R + r + 1], (BP, HH))],
                    axis=1)
                term = p_ref[c, r] * scale
                acc = term if acc is None else acc + term
            aggp.append(acc)
        agg_e = jnp.concatenate([aggp[0][:, :HH], aggp[1][:, :HH]], axis=1)
        agg_o = jnp.concatenate([aggp[0][:, HH:], aggp[1][:, HH:]], axis=1)
        xb = x_ref[...]
        wr = wr_ref[...]
        b = b1_ref[...]
        h_e = jnp.maximum(agg_e + jnp.dot(xb[:, :D], wr,
                          preferred_element_type=jnp.float32) + b, 0.0)
        h_o = jnp.maximum(agg_o + jnp.dot(xb[:, D:], wr,
                          preferred_element_type=jnp.float32) + b, 0.0)
        oh_ref[...] = jnp.concatenate([h_e, h_o], axis=1)
        wn = wn_ref[...]
        ohw_ref[...] = jnp.concatenate(
            [jnp.dot(h_e, wn, preferred_element_type=jnp.float32),
             jnp.dot(h_o, wn, preferred_element_type=jnp.float32)], axis=1)

    return pl.pallas_call(
        body,
        grid=(5,),
        in_specs=[pl.BlockSpec((NC, R, BP, H), lambda i: (0, 0, i, 0)),
                  pl.BlockSpec((NW, BP, 2 * R), lambda i: (0, i, 0)),
                  pl.BlockSpec((BP, 2 * D), lambda i: (i, 0)),
                  pl.BlockSpec((D, H), lambda i: (0, 0)),
                  pl.BlockSpec((1, H), lambda i: (0, 0)),
                  pl.BlockSpec((H, H), lambda i: (0, 0))],
        out_specs=[pl.BlockSpec((BP, 2 * H), lambda i: (i, 0)),
                   pl.BlockSpec((BP, 2 * H), lambda i: (i, 0))],
        out_shape=[jax.ShapeDtypeStruct((NP2, 2 * H), jnp.float32),
                   jax.ShapeDtypeStruct((NP2, 2 * H), jnp.float32)],
    )(parts, hist, xp, W_root, b1, Wg_nbr)


def _tc_post(xp, hp, parts2, Wg_root, b2):
    def body(x_ref, h_ref, q_ref, wg_ref, b2_ref, o_ref):
        q = q_ref[0] + q_ref[1]
        hb = h_ref[...]
        wg = wg_ref[...]
        b = b2_ref[...]
        h2_e = jnp.maximum(jnp.dot(hb[:, :H], wg,
                           preferred_element_type=jnp.float32)
                           + q[:, :H] + b, 0.0)
        h2_o = jnp.maximum(jnp.dot(hb[:, H:], wg,
                           preferred_element_type=jnp.float32)
                           + q[:, H:] + b, 0.0)
        xb = x_ref[...]
        o_ref[...] = jnp.concatenate([xb[:, :D], h2_e, xb[:, D:], h2_o],
                                     axis=1)

    return pl.pallas_call(
        body,
        grid=(5,),
        in_specs=[pl.BlockSpec((BP, 2 * D), lambda i: (i, 0)),
                  pl.BlockSpec((BP, 2 * H), lambda i: (i, 0)),
                  pl.BlockSpec((NC, BP, 2 * H), lambda i: (0, i, 0)),
                  pl.BlockSpec((H, H), lambda i: (0, 0)),
                  pl.BlockSpec((1, H), lambda i: (0, 0))],
        out_specs=pl.BlockSpec((BP, 2 * (D + H)), lambda i: (i, 0)),
        out_shape=jax.ShapeDtypeStruct((NP2, 2 * (D + H)), jnp.float32),
    )(xp, hp, parts2, Wg_root, b2)


def _sc_pass1(table, gsrc2, gdst, hidx, zrows, zhist):
    n_rows = R * N
    rpt = n_rows // NS

    @functools.partial(
        pl.kernel,
        out_type=(jax.ShapeDtypeStruct((NC, n_rows, HH), jnp.float32),
                  jax.ShapeDtypeStruct((NW * n_rows,), jnp.float32)),
        mesh=plsc.VectorSubcoreMesh(**_MESH),
        compiler_params=_SC_PARAMS,
        scratch_types=[
            pltpu.VMEM_SHARED((n_rows + NS, HH), jnp.float32),
            pltpu.VMEM((8, CH), jnp.int32),
            pltpu.VMEM((8, CH), jnp.int32),
            pltpu.VMEM((8, CH), jnp.int32),
            pltpu.VMEM((4, CH, HH), jnp.float32),
            pltpu.VMEM((n_rows + NS,), jnp.float32),
            pltpu.SemaphoreType.DMA((8,)),
            pltpu.SemaphoreType.DMA((4,)),
            pltpu.SemaphoreType.DMA((4,)),
        ],
    )
    def k(tab_hbm, gsrc_hbm, gdst_hbm, hidx_hbm, z_hbm, zh_hbm,
          parts_hbm, hist_hbm,
          acc, si_v, di_v, hi_v, rows_v, hist_v, sem_i, sem_g, sem_s):
        c = lax.axis_index("c")
        s = lax.axis_index("s")
        wid = c * NS + s
        gbase = (c * NS + s) * EPT1
        dbase = s * EPT1

        def idx_start(j, q):
            pltpu.async_copy(gsrc_hbm.at[pl.ds(gbase + j * CH, CH)],
                             si_v.at[q], sem_i.at[q])
            pltpu.async_copy(gdst_hbm.at[pl.ds(dbase + j * CH, CH)],
                             di_v.at[q], sem_i.at[q])
            pltpu.async_copy(hidx_hbm.at[pl.ds(dbase + j * CH, CH)],
                             hi_v.at[q], sem_i.at[q])

        def idx_wait(j, q):
            pltpu.make_async_copy(gsrc_hbm.at[pl.ds(gbase + j * CH, CH)],
                                  si_v.at[q], sem_i.at[q]).wait()
            pltpu.make_async_copy(gdst_hbm.at[pl.ds(dbase + j * CH, CH)],
                                  di_v.at[q], sem_i.at[q]).wait()
            pltpu.make_async_copy(hidx_hbm.at[pl.ds(dbase + j * CH, CH)],
                                  hi_v.at[q], sem_i.at[q]).wait()

        def scatter_start(rq, iq):
            pltpu.async_copy(rows_v.at[rq], acc.at[di_v.at[iq]],
                             sem_s.at[rq], add=True)

        def scatter_wait(rq, iq):
            pltpu.make_async_copy(rows_v.at[rq], acc.at[di_v.at[iq]],
                                  sem_s.at[rq]).wait()

        def gather_start(rq, iq):
            pltpu.async_copy(tab_hbm.at[si_v.at[iq]], rows_v.at[rq],
                             sem_g.at[rq])

        def gather_wait(rq, iq):
            pltpu.make_async_copy(tab_hbm.at[si_v.at[iq]], rows_v.at[rq],
                                  sem_g.at[rq]).wait()

        ones = jnp.ones((16,), jnp.float32)

        def hist_add(q):
            for g in range(CH // 16):
                plsc.addupdate_scatter(hist_v,
                                       [hi_v[q, pl.ds(g * 16, 16)]], ones)

        pltpu.sync_copy(z_hbm, acc.at[pl.ds(s * rpt, rpt)])
        pltpu.sync_copy(zh_hbm, hist_v)
        idx_start(0, 0)
        idx_start(1, 1)
        plsc.subcore_barrier()

        def octet(t, carry):
            for b in range(8):
                j = 8 * t + b
                rb = b % 4

                @pl.when(j >= 4)
                def _():
                    scatter_wait(rb, (b + 4) % 8)

                @pl.when(j + 2 < NCH1)
                def _():
                    idx_start(j + 2, (b + 2) % 8)

                @pl.when(j >= 1)
                def _():
                    gather_wait((rb + 3) % 4, (b + 7) % 8)
                    scatter_start((rb + 3) % 4, (b + 7) % 8)

                idx_wait(j, b)
                gather_start(rb, b)
                if b == 0:
                    @pl.when(j >= 1)
                    def _():
                        hist_add(7)
                else:
                    hist_add(b - 1)
            return carry

        lax.fori_loop(0, NCH1 // 8, octet, 0)
        gather_wait(3, 7)
        scatter_start(3, 7)
        hist_add(7)
        for kk in range(4):
            scatter_wait(kk, 4 + kk)
        plsc.subcore_barrier()
        pltpu.sync_copy(acc.at[pl.ds(s * rpt, rpt)],
                        parts_hbm.at[c, pl.ds(s * rpt, rpt)])
        pltpu.sync_copy(hist_v.at[pl.ds(0, n_rows)],
                        hist_hbm.at[pl.ds(wid * n_rows, n_rows)])

    return k(table, gsrc2, gdst, hidx, zrows, zhist)


def _sc_pass2(table, gsrc, gdst, zrows):
    n_rows = N
    rpt = n_rows // NS

    @functools.partial(
        pl.kernel,
        out_type=jax.ShapeDtypeStruct((NC, n_rows, H), jnp.float32),
        mesh=plsc.VectorSubcoreMesh(**_MESH),
        compiler_params=_SC_PARAMS,
        scratch_types=[
            pltpu.VMEM_SHARED((n_rows + NS, H), jnp.float32),
            pltpu.VMEM((8, CH), jnp.int32),
            pltpu.VMEM((8, CH), jnp.int32),
            pltpu.VMEM((4, CH, H), jnp.float32),
            pltpu.SemaphoreType.DMA((8,)),
            pltpu.SemaphoreType.DMA((4,)),
            pltpu.SemaphoreType.DMA((4,)),
        ],
    )
    def k(tab_hbm, gsrc_hbm, gdst_hbm, z_hbm, parts_hbm,
          acc, si_v, di_v, rows_v, sem_i, sem_g, sem_s):
        c = lax.axis_index("c")
        s = lax.axis_index("s")
        base = (c * NS + s) * EPW2

        def idx_start(j, q):
            pltpu.async_copy(gsrc_hbm.at[pl.ds(base + j * CH, CH)],
                             si_v.at[q], sem_i.at[q])
            pltpu.async_copy(gdst_hbm.at[pl.ds(base + j * CH, CH)],
                             di_v.at[q], sem_i.at[q])

        def idx_wait(j, q):
            pltpu.make_async_copy(gsrc_hbm.at[pl.ds(base + j * CH, CH)],
                                  si_v.at[q], sem_i.at[q]).wait()
            pltpu.make_async_copy(gdst_hbm.at[pl.ds(base + j * CH, CH)],
                                  di_v.at[q], sem_i.at[q]).wait()

        def scatter_start(rq, iq):
            pltpu.async_copy(rows_v.at[rq], acc.at[di_v.at[iq]],
                             sem_s.at[rq], add=True)

        def scatter_wait(rq, iq):
            pltpu.make_async_copy(rows_v.at[rq], acc.at[di_v.at[iq]],
                                  sem_s.at[rq]).wait()

        def gather_start(rq, iq):
            pltpu.async_copy(tab_hbm.at[si_v.at[iq]], rows_v.at[rq],
                             sem_g.at[rq])

        def gather_wait(rq, iq):
            pltpu.make_async_copy(tab_hbm.at[si_v.at[iq]], rows_v.at[rq],
                                  sem_g.at[rq]).wait()

        pltpu.sync_copy(z_hbm, acc.at[pl.ds(s * rpt, rpt)])
        idx_start(0, 0)
        idx_start(1, 1)
        plsc.subcore_barrier()

        def octet(t, carry):
            for b in range(8):
                j = 8 * t + b
                rb = b % 4

                @pl.when(j >= 4)
                def _():
                    scatter_wait(rb, (b + 4) % 8)

                @pl.when(j + 2 < NCH2)
                def _():
                    idx_start(j + 2, (b + 2) % 8)

                @pl.when(j >= 1)
                def _():
                    gather_wait((rb + 3) % 4, (b + 7) % 8)
                    scatter_start((rb + 3) % 4, (b + 7) % 8)

                idx_wait(j, b)
                gather_start(rb, b)
            return carry

        lax.fori_loop(0, NCH2 // 8, octet, 0)
        gather_wait(3, 7)
        scatter_start(3, 7)
        for kk in range(4):
            scatter_wait(kk, 4 + kk)
        plsc.subcore_barrier()
        pltpu.sync_copy(acc.at[pl.ds(s * rpt, rpt)],
                        parts_hbm.at[c, pl.ds(s * rpt, rpt)])

    return k(table, gsrc, gdst, zrows)


def kernel(x, edge_index, edge_type, W_rel, W_root, b1, Wg_root, Wg_nbr, b2):
    src = edge_index[0]
    dst = edge_index[1]

    def pad16(idx, trash):
        ppw = EPT1 - E // NS
        if trash is None:
            fill = jnp.zeros((NS, ppw), jnp.int32)
        else:
            fill = jnp.broadcast_to(trash + jnp.arange(NS)[:, None],
                                    (NS, ppw)).astype(jnp.int32)
        return jnp.concatenate([idx.reshape(NS, E // NS), fill],
                               axis=1).ravel()

    def pad32(idx, trash):
        ppw = EPW2 - E // NW
        if trash is None:
            fill = jnp.zeros((NW, ppw), jnp.int32)
        else:
            fill = jnp.broadcast_to(
                trash + (jnp.arange(NW)[:, None] % NS),
                (NW, ppw)).astype(jnp.int32)
        return jnp.concatenate([idx.reshape(NW, E // NW), fill],
                               axis=1).ravel()

    gsrc1 = pad16(edge_type * N + src, None)
    gsrc2 = jnp.concatenate([gsrc1, gsrc1 + R * N])
    gdst1 = pad16(edge_type * N + dst, R * N)
    hidx1 = pad16(dst * R + edge_type, R * N)
    src_p = pad32(src, None)
    dst_p = pad32(dst, N)

    zrows1 = jnp.zeros((R * N // NS, HH), jnp.float32)
    zrows2 = jnp.zeros((N // NS, H), jnp.float32)
    zhist = jnp.zeros((R * N + NS,), jnp.float32)

    xp = x.reshape(NP2, 2 * D)
    xrp = _tc_rel_transform(xp, W_rel)
    parts, hist = _sc_pass1(xrp.reshape(NC * R * N, HH), gsrc2, gdst1,
                            hidx1, zrows1, zhist)
    hp, hwp = _tc_mid(parts.reshape(NC, R, NP2, H),
                      hist.reshape(NW, NP2, 2 * R),
                      xp, W_root, b1.reshape(1, H), Wg_nbr)
    parts2 = _sc_pass2(hwp.reshape(N, H), src_p, dst_p, zrows2)
    out = _tc_post(xp, hp, parts2.reshape(NC, NP2, 2 * H), Wg_root,
                   b2.reshape(1, H))
    return out.reshape(N, D + H)
